# baseline (speedup 1.0000x reference)
import functools

import jax
import jax.numpy as jnp
from jax import lax
from jax.experimental import pallas as pl
from jax.experimental.pallas import tpu as pltpu
from jax.experimental.pallas import tpu_sc as plsc

N = 50000
E = 800000
B = 512
NF = 64
NG = 50
GAMMA = 10.0

NPAD = 50176
EPAD = 819200
NB = 512
EB = 1024
C = 128
TRASH = N

NC, NS = 2, 16
LOG2 = 0.6931471805599453


def _bdot(a, b):
    return jnp.dot(a.astype(jnp.bfloat16), b.astype(jnp.bfloat16),
                   preferred_element_type=jnp.float32)


def _ssp(x):
    return jnp.log(1.0 + jnp.exp(-jnp.abs(x))) + jnp.maximum(x, 0.0) - LOG2


def _emb_kernel(z_ref, emb_ref, w_ref, b_ref, x_ref, m_ref):
    z = z_ref[...]
    onehot = (z == lax.broadcasted_iota(jnp.int32, (1, 100), 1)).astype(jnp.float32)
    x = jnp.dot(onehot, emb_ref[...], preferred_element_type=jnp.float32,
                precision=lax.Precision.HIGHEST)
    x_ref[...] = x
    m = _bdot(x, w_ref[...]) + b_ref[...]
    m_ref[0] = m[:, :32]
    m_ref[1] = m[:, 32:]


def _g_kernel(d2_ref, u_ref, w1_ref, b1_ref, w2_ref, b2_ref, ga_ref, gb_ref):
    d = jnp.sqrt(d2_ref[...] + 1e-12)
    rbf = jnp.exp(-GAMMA * jnp.square(d - u_ref[...]))
    g = _ssp(_bdot(rbf, w1_ref[...]) + b1_ref[...])
    g = _ssp(_bdot(g, w2_ref[...]) + b2_ref[...])
    ga_ref[...] = g[:, :32]
    gb_ref[...] = g[:, 32:]


def _node_kernel(v_ref, x_ref, w1_ref, b1_ref, w2_ref, b2_ref,
                 nw_ref, nb_ref, x2_ref, m2_ref):
    v = jnp.concatenate([v_ref[0], v_ref[1]], axis=1)
    h = _ssp(_bdot(v, w1_ref[...]) + b1_ref[...])
    h = _bdot(h, w2_ref[...]) + b2_ref[...]
    x2 = x_ref[...] + h
    x2_ref[...] = x2
    m2 = _bdot(x2, nw_ref[...]) + nb_ref[...]
    m2_ref[0] = m2[:, :32]
    m2_ref[1] = m2[:, 32:]


def _post_kernel(v_ref, x_ref, w1_ref, b1_ref, w2_ref, b2_ref,
                 pw1_ref, pb1_ref, pw2_ref, pb2_ref, p_ref):
    v = jnp.concatenate([v_ref[0], v_ref[1]], axis=1)
    h = _ssp(_bdot(v, w1_ref[...]) + b1_ref[...])
    h = _bdot(h, w2_ref[...]) + b2_ref[...]
    x3 = x_ref[...] + h
    p = _ssp(_bdot(x3, pw1_ref[...]) + pb1_ref[...])
    p_ref[...] = _bdot(p, pw2_ref[...]) + pb2_ref[...]


def _head_kernel(p_ref, batch_ref, solv_ref, embs_ref, sw1_ref, sb1_ref,
                 sw2_ref, sb2_ref, qw1a_ref, qw1b_ref, qb1_ref,
                 qw2_ref, qb2_ref, qw3_ref, qb3_ref, out_ref,
                 acc_ref, cnt_ref):
    i = pl.program_id(0)

    @pl.when(i == 0)
    def _init():
        acc_ref[...] = jnp.zeros_like(acc_ref)
        cnt_ref[...] = jnp.zeros_like(cnt_ref)

    onehot = (lax.broadcasted_iota(jnp.int32, (B, 1), 0) == batch_ref[...]
              ).astype(jnp.float32)
    acc_ref[...] += jnp.dot(onehot, p_ref[...], preferred_element_type=jnp.float32,
                            precision=lax.Precision.HIGHEST)
    cnt_ref[...] += jnp.sum(onehot, axis=1, keepdims=True)

    @pl.when(i == NPAD // NB - 1)
    def _final():
        xg = acc_ref[...] / jnp.maximum(cnt_ref[...], 1.0)
        oh = (solv_ref[...] == lax.broadcasted_iota(jnp.int32, (1, 4), 1)
              ).astype(jnp.float32)
        s = jnp.dot(oh, embs_ref[...], preferred_element_type=jnp.float32,
                    precision=lax.Precision.HIGHEST)
        s = _ssp(_bdot(s, sw1_ref[...]) + sb1_ref[...])
        s = _bdot(s, sw2_ref[...]) + sb2_ref[...]
        h = _ssp(_bdot(xg, qw1a_ref[...])
                 + _bdot(s, qw1b_ref[...])
                 + qb1_ref[...])
        h = _ssp(_bdot(h, qw2_ref[...]) + qb2_ref[...])
        out_ref[...] = _bdot(h, qw3_ref[...]) + qb3_ref[...]


def _full(shape):
    return pl.BlockSpec(shape, lambda i: tuple(0 for _ in shape))


_MESH = plsc.VectorSubcoreMesh(core_axis_name="c", subcore_axis_name="s",
                               num_cores=NC, num_subcores=NS)


@functools.partial(
    pl.kernel,
    out_type=jax.ShapeDtypeStruct((EPAD,), jnp.float32),
    mesh=_MESH,
    scratch_types=[
        pltpu.VMEM((C,), jnp.int32),
        pltpu.VMEM((C,), jnp.int32),
        pltpu.VMEM((C, 16), jnp.float32),
        pltpu.VMEM((C, 16), jnp.float32),
        pltpu.VMEM((C,), jnp.float32),
        pltpu.SemaphoreType.DMA,
        pltpu.SemaphoreType.DMA,
    ],
    compiler_params=pltpu.CompilerParams(needs_layout_passes=False,
                                         use_tc_tiling_on_sc=False),
)
def _sc_dist(pos_hbm, src_hbm, dst_hbm, d2_hbm,
             sidx, didx, ps, pd, d2v, sem1, sem2):
    c = lax.axis_index("c")
    s = lax.axis_index("s")
    wid = s * NC + c
    per_w = EPAD // (NC * NS)
    nchunks = per_w // C

    def chunk(j, _):
        base = wid * per_w + j * C
        pltpu.sync_copy(src_hbm.at[pl.ds(base, C)], sidx)
        pltpu.sync_copy(dst_hbm.at[pl.ds(base, C)], didx)
        cp1 = pltpu.async_copy(pos_hbm.at[sidx], ps, sem1)
        cp2 = pltpu.async_copy(pos_hbm.at[didx], pd, sem2)
        cp1.wait()
        cp2.wait()
        lanes = lax.iota(jnp.int32, 16)
        for i in range(C // 16):
            rows = lanes + (i * 16)
            c0 = jnp.zeros((16,), jnp.int32)
            c1 = jnp.full((16,), 1, jnp.int32)
            c2 = jnp.full((16,), 2, jnp.int32)
            dx = (plsc.load_gather(ps, [rows, c0])
                  - plsc.load_gather(pd, [rows, c0]))
            dy = (plsc.load_gather(ps, [rows, c1])
                  - plsc.load_gather(pd, [rows, c1]))
            dz = (plsc.load_gather(ps, [rows, c2])
                  - plsc.load_gather(pd, [rows, c2]))
            d2v[pl.ds(i * 16, 16)] = dx * dx + dy * dy + dz * dz
        pltpu.sync_copy(d2v, d2_hbm.at[pl.ds(base, C)])
        return 0

    lax.fori_loop(0, nchunks, chunk, 0)


@functools.partial(
    pl.kernel,
    out_type=jax.ShapeDtypeStruct((2 * NPAD, 32), jnp.float32),
    mesh=_MESH,
    scratch_types=[
        pltpu.VMEM((2, C), jnp.int32),
        pltpu.VMEM((2, C), jnp.int32),
        pltpu.VMEM((C, 32), jnp.float32),
        pltpu.VMEM((C, 32), jnp.float32),
        pltpu.VMEM((2 * C, 32), jnp.float32),
        pltpu.VMEM((2 * C, 32), jnp.float32),
        pltpu.VMEM_SHARED((NPAD, 32), jnp.float32),
        pltpu.SemaphoreType.DMA,
        pltpu.SemaphoreType.DMA,
    ],
    compiler_params=pltpu.CompilerParams(use_tc_tiling_on_sc=False),
)
def _sc_msg(m_hbm, ga_hbm, gb_hbm, src_hbm, dst_hbm, zeros_hbm, v_hbm,
            sidx, didx, rows0, rows1, gv, msg, acc, gsem, isem):
    c = lax.axis_index("c")
    s = lax.axis_index("s")
    stripe = NPAD // NS
    per_s = EPAD // NS
    nchunks = per_s // (2 * C)

    pltpu.sync_copy(zeros_hbm.at[pl.ds(s * stripe, stripe)],
                    acc.at[pl.ds(s * stripe, stripe)])
    plsc.subcore_barrier()

    def chunk(j, _):
        base = s * per_s + j * (2 * C)
        row = base // C
        ci1 = pltpu.async_copy(src_hbm.at[pl.ds(c * (EPAD // C) + row, 2)], sidx, isem)
        ci2 = pltpu.async_copy(dst_hbm.at[pl.ds(row, 2)], didx, isem)
        ci1.wait()
        ci2.wait()
        cp0 = pltpu.async_copy(m_hbm.at[sidx.at[0]], rows0, gsem)
        cp1 = pltpu.async_copy(m_hbm.at[sidx.at[1]], rows1, gsem)

        @pl.when(c == 0)
        def _ga():
            pltpu.sync_copy(ga_hbm.at[pl.ds(base, 2 * C)], gv)

        @pl.when(c == 1)
        def _gb():
            pltpu.sync_copy(gb_hbm.at[pl.ds(base, 2 * C)], gv)

        cp0.wait()
        cp1.wait()

        def mrow(i, _):
            msg[i, 0:16] = rows0[i, 0:16] * gv[i, 0:16]
            msg[i, 16:32] = rows0[i, 16:32] * gv[i, 16:32]
            k = i + C
            msg[k, 0:16] = rows1[i, 0:16] * gv[k, 0:16]
            msg[k, 16:32] = rows1[i, 16:32] * gv[k, 16:32]
            return 0

        lax.fori_loop(0, C, mrow, 0, unroll=4)
        pltpu.sync_copy(msg.at[pl.ds(0, C)], acc.at[didx.at[0]], add=True)
        pltpu.sync_copy(msg.at[pl.ds(C, C)], acc.at[didx.at[1]], add=True)
        return 0

    lax.fori_loop(0, nchunks, chunk, 0)
    plsc.subcore_barrier()
    pltpu.sync_copy(acc.at[pl.ds(s * stripe, stripe)],
                    v_hbm.at[pl.ds(c * NPAD + s * stripe, stripe)])


def kernel(position, params, z, edge_index, batch, solvent):
    p = params
    f32 = jnp.float32

    pos_pad = jnp.zeros((NPAD, 16), f32).at[:N, :3].set(position)
    src = edge_index[0].astype(jnp.int32)
    dst = edge_index[1].astype(jnp.int32)
    src_pad = jnp.concatenate([src, jnp.zeros((EPAD - E,), jnp.int32)])
    dst_pad = jnp.concatenate([dst, jnp.full((EPAD - E,), TRASH, jnp.int32)])
    src2 = jnp.concatenate([src_pad, src_pad + NPAD]).reshape(2 * EPAD // C, C)
    dst2 = dst_pad.reshape(EPAD // C, C)
    zeros_nodes = jnp.zeros((NPAD, 32), f32)
    z_pad = jnp.concatenate([z.astype(jnp.int32), jnp.zeros((NPAD - N,), jnp.int32)])
    z2d = z_pad.reshape(NPAD, 1)
    batch_row = jnp.concatenate([batch.astype(jnp.int32),
                                 jnp.full((NPAD - N,), B, jnp.int32)]).reshape(1, NPAD)
    solv2d = solvent.astype(jnp.int32).reshape(B, 1)
    u_k = jnp.arange(0.0, 5.0, 0.1, dtype=f32).reshape(1, NG)

    ngrid = NPAD // NB
    egrid = EPAD // EB

    x, m1 = pl.pallas_call(
        _emb_kernel,
        grid=(ngrid,),
        in_specs=[pl.BlockSpec((NB, 1), lambda i: (i, 0)),
                  _full((100, NF)), _full((NF, NF)), _full((1, NF))],
        out_specs=[pl.BlockSpec((NB, NF), lambda i: (i, 0)),
                   pl.BlockSpec((2, NB, 32), lambda i: (0, i, 0))],
        out_shape=[jax.ShapeDtypeStruct((NPAD, NF), f32),
                   jax.ShapeDtypeStruct((2, NPAD, 32), f32)],
    )(z2d, p['emb_z'], p['inter'][0]['lin1_w'],
      p['inter'][0]['lin1_b'].reshape(1, NF))

    d2 = _sc_dist(pos_pad, src_pad, dst_pad)

    wspecs = [_full((1, NG)), _full((NG, NF)), _full((1, NF)), _full((NF, NF)),
              _full((1, NF))]
    d2c = d2.reshape(EPAD, 1)

    def g_call(lp):
        return pl.pallas_call(
            _g_kernel,
            grid=(egrid,),
            in_specs=[pl.BlockSpec((EB, 1), lambda i: (i, 0))] + wspecs,
            out_specs=[pl.BlockSpec((EB, 32), lambda i: (i, 0)),
                       pl.BlockSpec((EB, 32), lambda i: (i, 0))],
            out_shape=[jax.ShapeDtypeStruct((EPAD, 32), f32),
                       jax.ShapeDtypeStruct((EPAD, 32), f32)],
        )(d2c, u_k, lp['g_w1'], lp['g_b1'].reshape(1, NF),
          lp['g_w2'], lp['g_b2'].reshape(1, NF))

    g1a, g1b = g_call(p['inter'][0])
    g2a, g2b = g_call(p['inter'][1])

    v1 = _sc_msg(m1.reshape(2 * NPAD, 32), g1a, g1b,
                 src2, dst2, zeros_nodes)
    v1 = v1.reshape(2, NPAD, 32)

    lp0, lp1 = p['inter'][0], p['inter'][1]
    x2, m2 = pl.pallas_call(
        _node_kernel,
        grid=(ngrid,),
        in_specs=[pl.BlockSpec((2, NB, 32), lambda i: (0, i, 0)),
                  pl.BlockSpec((NB, NF), lambda i: (i, 0)),
                  _full((NF, NF)), _full((1, NF)), _full((NF, NF)), _full((1, NF)),
                  _full((NF, NF)), _full((1, NF))],
        out_specs=[pl.BlockSpec((NB, NF), lambda i: (i, 0)),
                   pl.BlockSpec((2, NB, 32), lambda i: (0, i, 0))],
        out_shape=[jax.ShapeDtypeStruct((NPAD, NF), f32),
                   jax.ShapeDtypeStruct((2, NPAD, 32), f32)],
    )(v1, x, lp0['m_w1'], lp0['m_b1'].reshape(1, NF),
      lp0['m_w2'], lp0['m_b2'].reshape(1, NF),
      lp1['lin1_w'], lp1['lin1_b'].reshape(1, NF))

    v2 = _sc_msg(m2.reshape(2 * NPAD, 32), g2a, g2b,
                 src2, dst2, zeros_nodes)
    v2 = v2.reshape(2, NPAD, 32)

    post = pl.pallas_call(
        _post_kernel,
        grid=(ngrid,),
        in_specs=[pl.BlockSpec((2, NB, 32), lambda i: (0, i, 0)),
                  pl.BlockSpec((NB, NF), lambda i: (i, 0)),
                  _full((NF, NF)), _full((1, NF)), _full((NF, NF)), _full((1, NF)),
                  _full((NF, NF)), _full((1, NF)), _full((NF, 64)), _full((1, 64))],
        out_specs=pl.BlockSpec((NB, 64), lambda i: (i, 0)),
        out_shape=jax.ShapeDtypeStruct((NPAD, 64), f32),
    )(v2, x2, lp1['m_w1'], lp1['m_b1'].reshape(1, NF),
      lp1['m_w2'], lp1['m_b2'].reshape(1, NF),
      p['post_w1'], p['post_b1'].reshape(1, NF),
      p['post_w2'], p['post_b2'].reshape(1, 64))

    out = pl.pallas_call(
        _head_kernel,
        grid=(ngrid,),
        in_specs=[pl.BlockSpec((NB, 64), lambda i: (i, 0)),
                  pl.BlockSpec((1, NB), lambda i: (0, i)),
                  _full((B, 1)), _full((4, 64)), _full((64, 64)), _full((1, 64)),
                  _full((64, 32)), _full((1, 32)),
                  _full((64, 128)), _full((32, 128)), _full((1, 128)),
                  _full((128, 32)), _full((1, 32)), _full((32, 1)), _full((1, 1))],
        out_specs=_full((B, 1)),
        out_shape=jax.ShapeDtypeStruct((B, 1), f32),
        scratch_shapes=[pltpu.VMEM((B, 64), f32), pltpu.VMEM((B, 1), f32)],
    )(post, batch_row, solv2d, p['emb_solv'],
      p['solv_w1'], p['solv_b1'].reshape(1, 64),
      p['solv_w2'], p['solv_b2'].reshape(1, 32),
      p['q_w1'][:64], p['q_w1'][64:], p['q_b1'].reshape(1, 128),
      p['q_w2'], p['q_b2'].reshape(1, 32),
      p['q_w3'], p['q_b3'].reshape(1, 1))

    return out


# speedup vs baseline: 1.0357x; 1.0357x over previous
import functools

import jax
import jax.numpy as jnp
from jax import lax
from jax.experimental import pallas as pl
from jax.experimental.pallas import tpu as pltpu
from jax.experimental.pallas import tpu_sc as plsc

N = 50000
E = 800000
B = 512
NF = 64
NG = 50
GAMMA = 10.0

NPAD = 50176
EPAD = 819200
NB = 512
EB = 1024
C = 128
TRASH = N

NC, NS = 2, 16
LOG2 = 0.6931471805599453


def _bdot(a, b):
    return jnp.dot(a.astype(jnp.bfloat16), b.astype(jnp.bfloat16),
                   preferred_element_type=jnp.float32)


def _ssp(x):
    return jnp.log(1.0 + jnp.exp(-jnp.abs(x))) + jnp.maximum(x, 0.0) - LOG2


def _emb_kernel(z_ref, emb_ref, w_ref, b_ref, x_ref, ma_ref, mb_ref):
    z = z_ref[...]
    onehot = (z == lax.broadcasted_iota(jnp.int32, (1, 100), 1)).astype(jnp.float32)
    x = jnp.dot(onehot, emb_ref[...], preferred_element_type=jnp.float32,
                precision=lax.Precision.HIGHEST)
    x_ref[...] = x
    m = _bdot(x, w_ref[...]) + b_ref[...]
    ma_ref[...] = m[:, :32]
    mb_ref[...] = m[:, 32:]


def _g_kernel(d2_ref, u_ref, w11_ref, b11_ref, w12_ref, b12_ref,
              w21_ref, b21_ref, w22_ref, b22_ref,
              g1a_ref, g1b_ref, g2a_ref, g2b_ref):
    d = jnp.sqrt(d2_ref[...] + 1e-12)
    rbf = jnp.exp(-GAMMA * jnp.square(d - u_ref[...]))
    g1 = _ssp(_bdot(rbf, w11_ref[...]) + b11_ref[...])
    g1 = _ssp(_bdot(g1, w12_ref[...]) + b12_ref[...])
    g2 = _ssp(_bdot(rbf, w21_ref[...]) + b21_ref[...])
    g2 = _ssp(_bdot(g2, w22_ref[...]) + b22_ref[...])
    g1a_ref[...] = g1[:, :32]
    g1b_ref[...] = g1[:, 32:]
    g2a_ref[...] = g2[:, :32]
    g2b_ref[...] = g2[:, 32:]


def _node_kernel(va_ref, vb_ref, x_ref, w1_ref, b1_ref, w2_ref, b2_ref,
                 nw_ref, nb_ref, x2_ref, ma_ref, mb_ref):
    v = jnp.concatenate([va_ref[...], vb_ref[...]], axis=1)
    h = _ssp(_bdot(v, w1_ref[...]) + b1_ref[...])
    h = _bdot(h, w2_ref[...]) + b2_ref[...]
    x2 = x_ref[...] + h
    x2_ref[...] = x2
    m2 = _bdot(x2, nw_ref[...]) + nb_ref[...]
    ma_ref[...] = m2[:, :32]
    mb_ref[...] = m2[:, 32:]


def _post_kernel(va_ref, vb_ref, x_ref, w1_ref, b1_ref, w2_ref, b2_ref,
                 pw1_ref, pb1_ref, pw2_ref, pb2_ref, p_ref):
    v = jnp.concatenate([va_ref[...], vb_ref[...]], axis=1)
    h = _ssp(_bdot(v, w1_ref[...]) + b1_ref[...])
    h = _bdot(h, w2_ref[...]) + b2_ref[...]
    x3 = x_ref[...] + h
    p = _ssp(_bdot(x3, pw1_ref[...]) + pb1_ref[...])
    p_ref[...] = _bdot(p, pw2_ref[...]) + pb2_ref[...]


def _head_kernel(p_ref, batch_ref, solv_ref, embs_ref, sw1_ref, sb1_ref,
                 sw2_ref, sb2_ref, qw1a_ref, qw1b_ref, qb1_ref,
                 qw2_ref, qb2_ref, qw3_ref, qb3_ref, out_ref,
                 acc_ref, cnt_ref):
    i = pl.program_id(0)

    @pl.when(i == 0)
    def _init():
        acc_ref[...] = jnp.zeros_like(acc_ref)
        cnt_ref[...] = jnp.zeros_like(cnt_ref)

    onehot = (lax.broadcasted_iota(jnp.int32, (B, 1), 0) == batch_ref[...]
              ).astype(jnp.float32)
    acc_ref[...] += jnp.dot(onehot, p_ref[...], preferred_element_type=jnp.float32,
                            precision=lax.Precision.HIGHEST)
    cnt_ref[...] += jnp.sum(onehot, axis=1, keepdims=True)

    @pl.when(i == NPAD // NB - 1)
    def _final():
        xg = acc_ref[...] / jnp.maximum(cnt_ref[...], 1.0)
        oh = (solv_ref[...] == lax.broadcasted_iota(jnp.int32, (1, 4), 1)
              ).astype(jnp.float32)
        s = jnp.dot(oh, embs_ref[...], preferred_element_type=jnp.float32,
                    precision=lax.Precision.HIGHEST)
        s = _ssp(_bdot(s, sw1_ref[...]) + sb1_ref[...])
        s = _bdot(s, sw2_ref[...]) + sb2_ref[...]
        h = _ssp(_bdot(xg, qw1a_ref[...])
                 + _bdot(s, qw1b_ref[...])
                 + qb1_ref[...])
        h = _ssp(_bdot(h, qw2_ref[...]) + qb2_ref[...])
        out_ref[...] = _bdot(h, qw3_ref[...]) + qb3_ref[...]


def _full(shape):
    return pl.BlockSpec(shape, lambda i: tuple(0 for _ in shape))


_MESH = plsc.VectorSubcoreMesh(core_axis_name="c", subcore_axis_name="s",
                               num_cores=NC, num_subcores=NS)


@functools.partial(
    pl.kernel,
    out_type=jax.ShapeDtypeStruct((EPAD,), jnp.float32),
    mesh=_MESH,
    scratch_types=[
        pltpu.VMEM((C,), jnp.int32),
        pltpu.VMEM((C,), jnp.int32),
        pltpu.VMEM((C, 16), jnp.float32),
        pltpu.VMEM((C, 16), jnp.float32),
        pltpu.VMEM((C,), jnp.float32),
        pltpu.SemaphoreType.DMA,
        pltpu.SemaphoreType.DMA,
    ],
    compiler_params=pltpu.CompilerParams(needs_layout_passes=False,
                                         use_tc_tiling_on_sc=False),
)
def _sc_dist(pos_hbm, src_hbm, dst_hbm, d2_hbm,
             sidx, didx, ps, pd, d2v, sem1, sem2):
    c = lax.axis_index("c")
    s = lax.axis_index("s")
    wid = s * NC + c
    per_w = EPAD // (NC * NS)
    nchunks = per_w // C

    def chunk(j, _):
        base = wid * per_w + j * C
        pltpu.sync_copy(src_hbm.at[pl.ds(base, C)], sidx)
        pltpu.sync_copy(dst_hbm.at[pl.ds(base, C)], didx)
        cp1 = pltpu.async_copy(pos_hbm.at[sidx], ps, sem1)
        cp2 = pltpu.async_copy(pos_hbm.at[didx], pd, sem2)
        cp1.wait()
        cp2.wait()
        lanes = lax.iota(jnp.int32, 16)
        for i in range(C // 16):
            rows = lanes + (i * 16)
            c0 = jnp.zeros((16,), jnp.int32)
            c1 = jnp.full((16,), 1, jnp.int32)
            c2 = jnp.full((16,), 2, jnp.int32)
            dx = (plsc.load_gather(ps, [rows, c0])
                  - plsc.load_gather(pd, [rows, c0]))
            dy = (plsc.load_gather(ps, [rows, c1])
                  - plsc.load_gather(pd, [rows, c1]))
            dz = (plsc.load_gather(ps, [rows, c2])
                  - plsc.load_gather(pd, [rows, c2]))
            d2v[pl.ds(i * 16, 16)] = dx * dx + dy * dy + dz * dz
        pltpu.sync_copy(d2v, d2_hbm.at[pl.ds(base, C)])
        return 0

    lax.fori_loop(0, nchunks, chunk, 0)


@functools.partial(
    pl.kernel,
    out_type=[jax.ShapeDtypeStruct((NPAD, 32), jnp.float32),
              jax.ShapeDtypeStruct((NPAD, 32), jnp.float32)],
    mesh=_MESH,
    scratch_types=[
        pltpu.VMEM((2, C), jnp.int32),
        pltpu.VMEM((2, C), jnp.int32),
        pltpu.VMEM((C, 32), jnp.float32),
        pltpu.VMEM((C, 32), jnp.float32),
        pltpu.VMEM((2 * C, 32), jnp.float32),
        pltpu.VMEM((2 * C, 32), jnp.float32),
        pltpu.VMEM_SHARED((NPAD, 32), jnp.float32),
        pltpu.SemaphoreType.DMA,
        pltpu.SemaphoreType.DMA,
    ],
    compiler_params=pltpu.CompilerParams(use_tc_tiling_on_sc=False),
)
def _sc_msg(ma_hbm, mb_hbm, ga_hbm, gb_hbm, src_hbm, dst_hbm, zeros_hbm,
            va_hbm, vb_hbm,
            sidx, didx, rows0, rows1, gv, msg, acc, gsem, isem):
    c = lax.axis_index("c")
    s = lax.axis_index("s")
    stripe = NPAD // NS
    per_s = EPAD // NS
    nchunks = per_s // (2 * C)

    pltpu.sync_copy(zeros_hbm.at[pl.ds(s * stripe, stripe)],
                    acc.at[pl.ds(s * stripe, stripe)])
    plsc.subcore_barrier()

    def chunk(j, _):
        base = s * per_s + j * (2 * C)
        row = base // C
        ci1 = pltpu.async_copy(src_hbm.at[pl.ds(row, 2)], sidx, isem)
        ci2 = pltpu.async_copy(dst_hbm.at[pl.ds(row, 2)], didx, isem)
        ci1.wait()
        ci2.wait()

        @pl.when(c == 0)
        def _ca():
            pltpu.async_copy(ma_hbm.at[sidx.at[0]], rows0, gsem)
            pltpu.async_copy(ma_hbm.at[sidx.at[1]], rows1, gsem)
            pltpu.sync_copy(ga_hbm.at[pl.ds(base, 2 * C)], gv)

        @pl.when(c == 1)
        def _cb():
            pltpu.async_copy(mb_hbm.at[sidx.at[0]], rows0, gsem)
            pltpu.async_copy(mb_hbm.at[sidx.at[1]], rows1, gsem)
            pltpu.sync_copy(gb_hbm.at[pl.ds(base, 2 * C)], gv)

        pltpu.make_async_copy(ma_hbm.at[sidx.at[0]], rows0, gsem).wait()
        pltpu.make_async_copy(ma_hbm.at[sidx.at[1]], rows1, gsem).wait()

        def mrow(i, _):
            msg[i, 0:16] = rows0[i, 0:16] * gv[i, 0:16]
            msg[i, 16:32] = rows0[i, 16:32] * gv[i, 16:32]
            k = i + C
            msg[k, 0:16] = rows1[i, 0:16] * gv[k, 0:16]
            msg[k, 16:32] = rows1[i, 16:32] * gv[k, 16:32]
            return 0

        lax.fori_loop(0, C, mrow, 0, unroll=4)
        pltpu.sync_copy(msg.at[pl.ds(0, C)], acc.at[didx.at[0]], add=True)
        pltpu.sync_copy(msg.at[pl.ds(C, C)], acc.at[didx.at[1]], add=True)
        return 0

    lax.fori_loop(0, nchunks, chunk, 0)
    plsc.subcore_barrier()

    @pl.when(c == 0)
    def _oa():
        pltpu.sync_copy(acc.at[pl.ds(s * stripe, stripe)],
                        va_hbm.at[pl.ds(s * stripe, stripe)])

    @pl.when(c == 1)
    def _ob():
        pltpu.sync_copy(acc.at[pl.ds(s * stripe, stripe)],
                        vb_hbm.at[pl.ds(s * stripe, stripe)])


def kernel(position, params, z, edge_index, batch, solvent):
    p = params
    f32 = jnp.float32

    pos_pad = jnp.zeros((NPAD, 16), f32).at[:N, :3].set(position)
    src = edge_index[0].astype(jnp.int32)
    dst = edge_index[1].astype(jnp.int32)
    src_pad = jnp.concatenate([src, jnp.zeros((EPAD - E,), jnp.int32)])
    dst_pad = jnp.concatenate([dst, jnp.full((EPAD - E,), TRASH, jnp.int32)])
    src2 = src_pad.reshape(EPAD // C, C)
    dst2 = dst_pad.reshape(EPAD // C, C)
    zeros_nodes = jnp.zeros((NPAD, 32), f32)
    z_pad = jnp.concatenate([z.astype(jnp.int32), jnp.zeros((NPAD - N,), jnp.int32)])
    z2d = z_pad.reshape(NPAD, 1)
    batch_row = jnp.concatenate([batch.astype(jnp.int32),
                                 jnp.full((NPAD - N,), B, jnp.int32)]).reshape(1, NPAD)
    solv2d = solvent.astype(jnp.int32).reshape(B, 1)
    u_k = jnp.arange(0.0, 5.0, 0.1, dtype=f32).reshape(1, NG)

    ngrid = NPAD // NB
    egrid = EPAD // EB

    nhalf = pl.BlockSpec((NB, 32), lambda i: (i, 0))
    sds_nh = jax.ShapeDtypeStruct((NPAD, 32), f32)

    x, m1a, m1b = pl.pallas_call(
        _emb_kernel,
        grid=(ngrid,),
        in_specs=[pl.BlockSpec((NB, 1), lambda i: (i, 0)),
                  _full((100, NF)), _full((NF, NF)), _full((1, NF))],
        out_specs=[pl.BlockSpec((NB, NF), lambda i: (i, 0)), nhalf, nhalf],
        out_shape=[jax.ShapeDtypeStruct((NPAD, NF), f32), sds_nh, sds_nh],
    )(z2d, p['emb_z'], p['inter'][0]['lin1_w'],
      p['inter'][0]['lin1_b'].reshape(1, NF))

    d2 = _sc_dist(pos_pad, src_pad, dst_pad)

    wspecs = [_full((1, NG)), _full((NG, NF)), _full((1, NF)), _full((NF, NF)),
              _full((1, NF)), _full((NG, NF)), _full((1, NF)), _full((NF, NF)),
              _full((1, NF))]
    ehalf = pl.BlockSpec((EB, 32), lambda i: (i, 0))
    sds_eh = jax.ShapeDtypeStruct((EPAD, 32), f32)
    g1a, g1b, g2a, g2b = pl.pallas_call(
        _g_kernel,
        grid=(egrid,),
        in_specs=[pl.BlockSpec((EB, 1), lambda i: (i, 0))] + wspecs,
        out_specs=[ehalf, ehalf, ehalf, ehalf],
        out_shape=[sds_eh, sds_eh, sds_eh, sds_eh],
    )(d2.reshape(EPAD, 1), u_k,
      p['inter'][0]['g_w1'], p['inter'][0]['g_b1'].reshape(1, NF),
      p['inter'][0]['g_w2'], p['inter'][0]['g_b2'].reshape(1, NF),
      p['inter'][1]['g_w1'], p['inter'][1]['g_b1'].reshape(1, NF),
      p['inter'][1]['g_w2'], p['inter'][1]['g_b2'].reshape(1, NF))

    v1a, v1b = _sc_msg(m1a, m1b, g1a, g1b, src2, dst2, zeros_nodes)

    lp0, lp1 = p['inter'][0], p['inter'][1]
    x2, m2a, m2b = pl.pallas_call(
        _node_kernel,
        grid=(ngrid,),
        in_specs=[nhalf, nhalf,
                  pl.BlockSpec((NB, NF), lambda i: (i, 0)),
                  _full((NF, NF)), _full((1, NF)), _full((NF, NF)), _full((1, NF)),
                  _full((NF, NF)), _full((1, NF))],
        out_specs=[pl.BlockSpec((NB, NF), lambda i: (i, 0)), nhalf, nhalf],
        out_shape=[jax.ShapeDtypeStruct((NPAD, NF), f32), sds_nh, sds_nh],
    )(v1a, v1b, x, lp0['m_w1'], lp0['m_b1'].reshape(1, NF),
      lp0['m_w2'], lp0['m_b2'].reshape(1, NF),
      lp1['lin1_w'], lp1['lin1_b'].reshape(1, NF))

    v2a, v2b = _sc_msg(m2a, m2b, g2a, g2b, src2, dst2, zeros_nodes)

    post = pl.pallas_call(
        _post_kernel,
        grid=(ngrid,),
        in_specs=[nhalf, nhalf,
                  pl.BlockSpec((NB, NF), lambda i: (i, 0)),
                  _full((NF, NF)), _full((1, NF)), _full((NF, NF)), _full((1, NF)),
                  _full((NF, NF)), _full((1, NF)), _full((NF, 64)), _full((1, 64))],
        out_specs=pl.BlockSpec((NB, 64), lambda i: (i, 0)),
        out_shape=jax.ShapeDtypeStruct((NPAD, 64), f32),
    )(v2a, v2b, x2, lp1['m_w1'], lp1['m_b1'].reshape(1, NF),
      lp1['m_w2'], lp1['m_b2'].reshape(1, NF),
      p['post_w1'], p['post_b1'].reshape(1, NF),
      p['post_w2'], p['post_b2'].reshape(1, 64))

    out = pl.pallas_call(
        _head_kernel,
        grid=(ngrid,),
        in_specs=[pl.BlockSpec((NB, 64), lambda i: (i, 0)),
                  pl.BlockSpec((1, NB), lambda i: (0, i)),
                  _full((B, 1)), _full((4, 64)), _full((64, 64)), _full((1, 64)),
                  _full((64, 32)), _full((1, 32)),
                  _full((64, 128)), _full((32, 128)), _full((1, 128)),
                  _full((128, 32)), _full((1, 32)), _full((32, 1)), _full((1, 1))],
        out_specs=_full((B, 1)),
        out_shape=jax.ShapeDtypeStruct((B, 1), f32),
        scratch_shapes=[pltpu.VMEM((B, 64), f32), pltpu.VMEM((B, 1), f32)],
    )(post, batch_row, solv2d, p['emb_solv'],
      p['solv_w1'], p['solv_b1'].reshape(1, 64),
      p['solv_w2'], p['solv_b2'].reshape(1, 32),
      p['q_w1'][:64], p['q_w1'][64:], p['q_b1'].reshape(1, 128),
      p['q_w2'], p['q_b2'].reshape(1, 32),
      p['q_w3'], p['q_b3'].reshape(1, 1))

    return out


# speedup vs baseline: 1.1249x; 1.0861x over previous
import functools

import jax
import jax.numpy as jnp
from jax import lax
from jax.experimental import pallas as pl
from jax.experimental.pallas import tpu as pltpu
from jax.experimental.pallas import tpu_sc as plsc

N = 50000
E = 800000
B = 512
NF = 64
NG = 50
GAMMA = 10.0

NPAD = 50176
EPAD = 819200
NB = 512
EB = 1024
C = 128
TRASH = N

NC, NS = 2, 16
LOG2 = 0.6931471805599453


def _bdot(a, b):
    return jnp.dot(a.astype(jnp.bfloat16), b.astype(jnp.bfloat16),
                   preferred_element_type=jnp.float32)


def _ssp(x):
    return jnp.log(1.0 + jnp.exp(-jnp.abs(x))) + jnp.maximum(x, 0.0) - LOG2


def _emb_kernel(z_ref, emb_ref, w_ref, b_ref, x_ref, ma_ref, mb_ref):
    z = z_ref[...]
    onehot = (z == lax.broadcasted_iota(jnp.int32, (1, 100), 1)).astype(jnp.float32)
    x = jnp.dot(onehot, emb_ref[...], preferred_element_type=jnp.float32,
                precision=lax.Precision.HIGHEST)
    x_ref[...] = x
    m = _bdot(x, w_ref[...]) + b_ref[...]
    ma_ref[...] = m[:, :32]
    mb_ref[...] = m[:, 32:]


def _g_kernel(d2_ref, u_ref, w11_ref, b11_ref, w12_ref, b12_ref,
              w21_ref, b21_ref, w22_ref, b22_ref,
              g1a_ref, g1b_ref, g2a_ref, g2b_ref):
    d = jnp.sqrt(d2_ref[...] + 1e-12).reshape(EB, 1)
    rbf = jnp.exp(-GAMMA * jnp.square(d - u_ref[...]))
    g1 = _ssp(_bdot(rbf, w11_ref[...]) + b11_ref[...])
    g1 = _ssp(_bdot(g1, w12_ref[...]) + b12_ref[...])
    g2 = _ssp(_bdot(rbf, w21_ref[...]) + b21_ref[...])
    g2 = _ssp(_bdot(g2, w22_ref[...]) + b22_ref[...])
    g1a_ref[...] = g1[:, :32]
    g1b_ref[...] = g1[:, 32:]
    g2a_ref[...] = g2[:, :32]
    g2b_ref[...] = g2[:, 32:]


def _node_kernel(va_ref, vb_ref, x_ref, w1_ref, b1_ref, w2_ref, b2_ref,
                 nw_ref, nb_ref, x2_ref, ma_ref, mb_ref):
    v = jnp.concatenate([va_ref[...], vb_ref[...]], axis=1)
    h = _ssp(_bdot(v, w1_ref[...]) + b1_ref[...])
    h = _bdot(h, w2_ref[...]) + b2_ref[...]
    x2 = x_ref[...] + h
    x2_ref[...] = x2
    m2 = _bdot(x2, nw_ref[...]) + nb_ref[...]
    ma_ref[...] = m2[:, :32]
    mb_ref[...] = m2[:, 32:]


def _post_kernel(va_ref, vb_ref, x_ref, w1_ref, b1_ref, w2_ref, b2_ref,
                 pw1_ref, pb1_ref, pw2_ref, pb2_ref, p_ref):
    v = jnp.concatenate([va_ref[...], vb_ref[...]], axis=1)
    h = _ssp(_bdot(v, w1_ref[...]) + b1_ref[...])
    h = _bdot(h, w2_ref[...]) + b2_ref[...]
    x3 = x_ref[...] + h
    p = _ssp(_bdot(x3, pw1_ref[...]) + pb1_ref[...])
    p_ref[...] = _bdot(p, pw2_ref[...]) + pb2_ref[...]


def _head_kernel(p_ref, batch_ref, solv_ref, embs_ref, sw1_ref, sb1_ref,
                 sw2_ref, sb2_ref, qw1a_ref, qw1b_ref, qb1_ref,
                 qw2_ref, qb2_ref, qw3_ref, qb3_ref, out_ref,
                 acc_ref, cnt_ref):
    i = pl.program_id(0)

    @pl.when(i == 0)
    def _init():
        acc_ref[...] = jnp.zeros_like(acc_ref)
        cnt_ref[...] = jnp.zeros_like(cnt_ref)

    onehot = (lax.broadcasted_iota(jnp.int32, (B, 1), 0) == batch_ref[...]
              ).astype(jnp.float32)
    acc_ref[...] += jnp.dot(onehot, p_ref[...], preferred_element_type=jnp.float32,
                            precision=lax.Precision.HIGHEST)
    cnt_ref[...] += jnp.sum(onehot, axis=1, keepdims=True)

    @pl.when(i == NPAD // NB - 1)
    def _final():
        xg = acc_ref[...] / jnp.maximum(cnt_ref[...], 1.0)
        oh = (solv_ref[...] == lax.broadcasted_iota(jnp.int32, (1, 4), 1)
              ).astype(jnp.float32)
        s = jnp.dot(oh, embs_ref[...], preferred_element_type=jnp.float32,
                    precision=lax.Precision.HIGHEST)
        s = _ssp(_bdot(s, sw1_ref[...]) + sb1_ref[...])
        s = _bdot(s, sw2_ref[...]) + sb2_ref[...]
        h = _ssp(_bdot(xg, qw1a_ref[...])
                 + _bdot(s, qw1b_ref[...])
                 + qb1_ref[...])
        h = _ssp(_bdot(h, qw2_ref[...]) + qb2_ref[...])
        out_ref[...] = _bdot(h, qw3_ref[...]) + qb3_ref[...]


def _full(shape):
    return pl.BlockSpec(shape, lambda i: tuple(0 for _ in shape))


_MESH = plsc.VectorSubcoreMesh(core_axis_name="c", subcore_axis_name="s",
                               num_cores=NC, num_subcores=NS)


@functools.partial(
    pl.kernel,
    out_type=jax.ShapeDtypeStruct((EPAD,), jnp.float32),
    mesh=_MESH,
    scratch_types=[
        pltpu.VMEM((C,), jnp.int32),
        pltpu.VMEM((C,), jnp.int32),
        pltpu.VMEM((C, 16), jnp.float32),
        pltpu.VMEM((C, 16), jnp.float32),
        pltpu.VMEM((C,), jnp.float32),
        pltpu.SemaphoreType.DMA,
        pltpu.SemaphoreType.DMA,
    ],
    compiler_params=pltpu.CompilerParams(needs_layout_passes=False,
                                         use_tc_tiling_on_sc=False),
)
def _sc_dist(pos_hbm, src_hbm, dst_hbm, d2_hbm,
             sidx, didx, ps, pd, d2v, sem1, sem2):
    c = lax.axis_index("c")
    s = lax.axis_index("s")
    wid = s * NC + c
    per_w = EPAD // (NC * NS)
    nchunks = per_w // C

    def chunk(j, _):
        base = wid * per_w + j * C
        pltpu.sync_copy(src_hbm.at[pl.ds(base, C)], sidx)
        pltpu.sync_copy(dst_hbm.at[pl.ds(base, C)], didx)
        cp1 = pltpu.async_copy(pos_hbm.at[sidx], ps, sem1)
        cp2 = pltpu.async_copy(pos_hbm.at[didx], pd, sem2)
        cp1.wait()
        cp2.wait()
        lanes = lax.iota(jnp.int32, 16)
        for i in range(C // 16):
            rows = lanes + (i * 16)
            c0 = jnp.zeros((16,), jnp.int32)
            c1 = jnp.full((16,), 1, jnp.int32)
            c2 = jnp.full((16,), 2, jnp.int32)
            dx = (plsc.load_gather(ps, [rows, c0])
                  - plsc.load_gather(pd, [rows, c0]))
            dy = (plsc.load_gather(ps, [rows, c1])
                  - plsc.load_gather(pd, [rows, c1]))
            dz = (plsc.load_gather(ps, [rows, c2])
                  - plsc.load_gather(pd, [rows, c2]))
            d2v[pl.ds(i * 16, 16)] = dx * dx + dy * dy + dz * dz
        pltpu.sync_copy(d2v, d2_hbm.at[pl.ds(base, C)])
        return 0

    lax.fori_loop(0, nchunks, chunk, 0)


@functools.partial(
    pl.kernel,
    out_type=[jax.ShapeDtypeStruct((NPAD, 32), jnp.float32),
              jax.ShapeDtypeStruct((NPAD, 32), jnp.float32)],
    mesh=_MESH,
    scratch_types=[
        pltpu.VMEM((C,), jnp.int32),
        pltpu.VMEM((C,), jnp.int32),
        pltpu.VMEM((C,), jnp.int32),
        pltpu.VMEM((C,), jnp.int32),
        pltpu.VMEM((C, 32), jnp.float32),
        pltpu.VMEM((C, 32), jnp.float32),
        pltpu.VMEM((2 * C, 32), jnp.float32),
        pltpu.VMEM((2 * C, 32), jnp.float32),
        pltpu.VMEM_SHARED((NPAD, 32), jnp.float32),
        pltpu.SemaphoreType.DMA,
        pltpu.SemaphoreType.DMA,
    ],
    compiler_params=pltpu.CompilerParams(use_tc_tiling_on_sc=False),
)
def _sc_msg(ma_hbm, mb_hbm, ga_hbm, gb_hbm, src_hbm, dst_hbm, zeros_hbm,
            va_hbm, vb_hbm,
            sidx0, sidx1, didx0, didx1, rows0, rows1, gv, msg, acc, gsem, isem):
    c = lax.axis_index("c")
    s = lax.axis_index("s")
    stripe = NPAD // NS
    per_s = EPAD // NS
    nchunks = per_s // (2 * C)

    pltpu.sync_copy(zeros_hbm.at[pl.ds(s * stripe, stripe)],
                    acc.at[pl.ds(s * stripe, stripe)])
    plsc.subcore_barrier()

    def chunk(j, _):
        base = s * per_s + j * (2 * C)
        ci1 = pltpu.async_copy(src_hbm.at[pl.ds(base, C)], sidx0, isem)
        ci2 = pltpu.async_copy(src_hbm.at[pl.ds(base + C, C)], sidx1, isem)
        ci3 = pltpu.async_copy(dst_hbm.at[pl.ds(base, C)], didx0, isem)
        ci4 = pltpu.async_copy(dst_hbm.at[pl.ds(base + C, C)], didx1, isem)
        ci1.wait()
        ci2.wait()
        ci3.wait()
        ci4.wait()

        @pl.when(c == 0)
        def _ca():
            pltpu.async_copy(ma_hbm.at[sidx0], rows0, gsem)
            pltpu.async_copy(ma_hbm.at[sidx1], rows1, gsem)
            pltpu.sync_copy(ga_hbm.at[pl.ds(base, 2 * C)], gv)

        @pl.when(c == 1)
        def _cb():
            pltpu.async_copy(mb_hbm.at[sidx0], rows0, gsem)
            pltpu.async_copy(mb_hbm.at[sidx1], rows1, gsem)
            pltpu.sync_copy(gb_hbm.at[pl.ds(base, 2 * C)], gv)

        pltpu.make_async_copy(ma_hbm.at[sidx0], rows0, gsem).wait()
        pltpu.make_async_copy(ma_hbm.at[sidx1], rows1, gsem).wait()

        def mrow(i, _):
            msg[i, 0:16] = rows0[i, 0:16] * gv[i, 0:16]
            msg[i, 16:32] = rows0[i, 16:32] * gv[i, 16:32]
            k = i + C
            msg[k, 0:16] = rows1[i, 0:16] * gv[k, 0:16]
            msg[k, 16:32] = rows1[i, 16:32] * gv[k, 16:32]
            return 0

        lax.fori_loop(0, C, mrow, 0, unroll=4)
        pltpu.sync_copy(msg.at[pl.ds(0, C)], acc.at[didx0], add=True)
        pltpu.sync_copy(msg.at[pl.ds(C, C)], acc.at[didx1], add=True)
        return 0

    lax.fori_loop(0, nchunks, chunk, 0)
    plsc.subcore_barrier()

    @pl.when(c == 0)
    def _oa():
        pltpu.sync_copy(acc.at[pl.ds(s * stripe, stripe)],
                        va_hbm.at[pl.ds(s * stripe, stripe)])

    @pl.when(c == 1)
    def _ob():
        pltpu.sync_copy(acc.at[pl.ds(s * stripe, stripe)],
                        vb_hbm.at[pl.ds(s * stripe, stripe)])


def kernel(position, params, z, edge_index, batch, solvent):
    p = params
    f32 = jnp.float32

    pos_pad = jnp.zeros((NPAD, 16), f32).at[:N, :3].set(position)
    src = edge_index[0].astype(jnp.int32)
    dst = edge_index[1].astype(jnp.int32)
    src_pad = jnp.concatenate([src, jnp.zeros((EPAD - E,), jnp.int32)])
    dst_pad = jnp.concatenate([dst, jnp.full((EPAD - E,), TRASH, jnp.int32)])
    zeros_nodes = jnp.zeros((NPAD, 32), f32)
    z_pad = jnp.concatenate([z.astype(jnp.int32), jnp.zeros((NPAD - N,), jnp.int32)])
    z2d = z_pad.reshape(NPAD, 1)
    batch_row = jnp.concatenate([batch.astype(jnp.int32),
                                 jnp.full((NPAD - N,), B, jnp.int32)]).reshape(1, NPAD)
    solv2d = solvent.astype(jnp.int32).reshape(B, 1)
    u_k = jnp.arange(0.0, 5.0, 0.1, dtype=f32).reshape(1, NG)

    ngrid = NPAD // NB
    egrid = EPAD // EB

    nhalf = pl.BlockSpec((NB, 32), lambda i: (i, 0))
    sds_nh = jax.ShapeDtypeStruct((NPAD, 32), f32)

    x, m1a, m1b = pl.pallas_call(
        _emb_kernel,
        grid=(ngrid,),
        in_specs=[pl.BlockSpec((NB, 1), lambda i: (i, 0)),
                  _full((100, NF)), _full((NF, NF)), _full((1, NF))],
        out_specs=[pl.BlockSpec((NB, NF), lambda i: (i, 0)), nhalf, nhalf],
        out_shape=[jax.ShapeDtypeStruct((NPAD, NF), f32), sds_nh, sds_nh],
    )(z2d, p['emb_z'], p['inter'][0]['lin1_w'],
      p['inter'][0]['lin1_b'].reshape(1, NF))

    d2 = _sc_dist(pos_pad, src_pad, dst_pad)

    wspecs = [_full((1, NG)), _full((NG, NF)), _full((1, NF)), _full((NF, NF)),
              _full((1, NF)), _full((NG, NF)), _full((1, NF)), _full((NF, NF)),
              _full((1, NF))]
    ehalf = pl.BlockSpec((EB, 32), lambda i: (i, 0))
    sds_eh = jax.ShapeDtypeStruct((EPAD, 32), f32)
    g1a, g1b, g2a, g2b = pl.pallas_call(
        _g_kernel,
        grid=(egrid,),
        in_specs=[pl.BlockSpec((EB,), lambda i: (i,))] + wspecs,
        out_specs=[ehalf, ehalf, ehalf, ehalf],
        out_shape=[sds_eh, sds_eh, sds_eh, sds_eh],
    )(d2, u_k,
      p['inter'][0]['g_w1'], p['inter'][0]['g_b1'].reshape(1, NF),
      p['inter'][0]['g_w2'], p['inter'][0]['g_b2'].reshape(1, NF),
      p['inter'][1]['g_w1'], p['inter'][1]['g_b1'].reshape(1, NF),
      p['inter'][1]['g_w2'], p['inter'][1]['g_b2'].reshape(1, NF))

    v1a, v1b = _sc_msg(m1a, m1b, g1a, g1b, src_pad, dst_pad, zeros_nodes)

    lp0, lp1 = p['inter'][0], p['inter'][1]
    x2, m2a, m2b = pl.pallas_call(
        _node_kernel,
        grid=(ngrid,),
        in_specs=[nhalf, nhalf,
                  pl.BlockSpec((NB, NF), lambda i: (i, 0)),
                  _full((NF, NF)), _full((1, NF)), _full((NF, NF)), _full((1, NF)),
                  _full((NF, NF)), _full((1, NF))],
        out_specs=[pl.BlockSpec((NB, NF), lambda i: (i, 0)), nhalf, nhalf],
        out_shape=[jax.ShapeDtypeStruct((NPAD, NF), f32), sds_nh, sds_nh],
    )(v1a, v1b, x, lp0['m_w1'], lp0['m_b1'].reshape(1, NF),
      lp0['m_w2'], lp0['m_b2'].reshape(1, NF),
      lp1['lin1_w'], lp1['lin1_b'].reshape(1, NF))

    v2a, v2b = _sc_msg(m2a, m2b, g2a, g2b, src_pad, dst_pad, zeros_nodes)

    post = pl.pallas_call(
        _post_kernel,
        grid=(ngrid,),
        in_specs=[nhalf, nhalf,
                  pl.BlockSpec((NB, NF), lambda i: (i, 0)),
                  _full((NF, NF)), _full((1, NF)), _full((NF, NF)), _full((1, NF)),
                  _full((NF, NF)), _full((1, NF)), _full((NF, 64)), _full((1, 64))],
        out_specs=pl.BlockSpec((NB, 64), lambda i: (i, 0)),
        out_shape=jax.ShapeDtypeStruct((NPAD, 64), f32),
    )(v2a, v2b, x2, lp1['m_w1'], lp1['m_b1'].reshape(1, NF),
      lp1['m_w2'], lp1['m_b2'].reshape(1, NF),
      p['post_w1'], p['post_b1'].reshape(1, NF),
      p['post_w2'], p['post_b2'].reshape(1, 64))

    out = pl.pallas_call(
        _head_kernel,
        grid=(ngrid,),
        in_specs=[pl.BlockSpec((NB, 64), lambda i: (i, 0)),
                  pl.BlockSpec((1, NB), lambda i: (0, i)),
                  _full((B, 1)), _full((4, 64)), _full((64, 64)), _full((1, 64)),
                  _full((64, 32)), _full((1, 32)),
                  _full((64, 128)), _full((32, 128)), _full((1, 128)),
                  _full((128, 32)), _full((1, 32)), _full((32, 1)), _full((1, 1))],
        out_specs=_full((B, 1)),
        out_shape=jax.ShapeDtypeStruct((B, 1), f32),
        scratch_shapes=[pltpu.VMEM((B, 64), f32), pltpu.VMEM((B, 1), f32)],
    )(post, batch_row, solv2d, p['emb_solv'],
      p['solv_w1'], p['solv_b1'].reshape(1, 64),
      p['solv_w2'], p['solv_b2'].reshape(1, 32),
      p['q_w1'][:64], p['q_w1'][64:], p['q_b1'].reshape(1, 128),
      p['q_w2'], p['q_b2'].reshape(1, 32),
      p['q_w3'], p['q_b3'].reshape(1, 1))

    return out


# speedup vs baseline: 1.2073x; 1.0733x over previous
import functools

import jax
import jax.numpy as jnp
from jax import lax
from jax.experimental import pallas as pl
from jax.experimental.pallas import tpu as pltpu
from jax.experimental.pallas import tpu_sc as plsc

N = 50000
E = 800000
B = 512
NF = 64
NG = 50
GAMMA = 10.0

NPAD = 50176
EPAD = 819200
NB = 512
EB = 1024
C = 128
TRASH = N

NC, NS = 2, 16
LOG2 = 0.6931471805599453


def _bdot(a, b):
    return jnp.dot(a.astype(jnp.bfloat16), b.astype(jnp.bfloat16),
                   preferred_element_type=jnp.float32)


def _ssp(x):
    return jnp.log(1.0 + jnp.exp(-jnp.abs(x))) + jnp.maximum(x, 0.0) - LOG2


def _emb_kernel(z_ref, emb_ref, w_ref, b_ref, x_ref, ma_ref, mb_ref):
    z = z_ref[...]
    onehot = (z == lax.broadcasted_iota(jnp.int32, (1, 100), 1)).astype(jnp.float32)
    x = jnp.dot(onehot, emb_ref[...], preferred_element_type=jnp.float32,
                precision=lax.Precision.HIGHEST)
    x_ref[...] = x
    m = _bdot(x, w_ref[...]) + b_ref[...]
    ma_ref[...] = m[:, :32]
    mb_ref[...] = m[:, 32:]


def _g_kernel(d2_ref, u_ref, w11_ref, b11_ref, w12_ref, b12_ref,
              w21_ref, b21_ref, w22_ref, b22_ref,
              g1a_ref, g1b_ref, g2a_ref, g2b_ref):
    d = jnp.sqrt(d2_ref[...] + 1e-12).reshape(EB, 1)
    rbf = jnp.exp(-GAMMA * jnp.square(d - u_ref[...]))
    g1 = _ssp(_bdot(rbf, w11_ref[...]) + b11_ref[...])
    g1 = _ssp(_bdot(g1, w12_ref[...]) + b12_ref[...])
    g2 = _ssp(_bdot(rbf, w21_ref[...]) + b21_ref[...])
    g2 = _ssp(_bdot(g2, w22_ref[...]) + b22_ref[...])
    g1a_ref[...] = g1[:, :32]
    g1b_ref[...] = g1[:, 32:]
    g2a_ref[...] = g2[:, :32]
    g2b_ref[...] = g2[:, 32:]


def _node_kernel(va_ref, vb_ref, x_ref, w1_ref, b1_ref, w2_ref, b2_ref,
                 nw_ref, nb_ref, x2_ref, ma_ref, mb_ref):
    v = jnp.concatenate([va_ref[...], vb_ref[...]], axis=1)
    h = _ssp(_bdot(v, w1_ref[...]) + b1_ref[...])
    h = _bdot(h, w2_ref[...]) + b2_ref[...]
    x2 = x_ref[...] + h
    x2_ref[...] = x2
    m2 = _bdot(x2, nw_ref[...]) + nb_ref[...]
    ma_ref[...] = m2[:, :32]
    mb_ref[...] = m2[:, 32:]


def _post_kernel(va_ref, vb_ref, x_ref, w1_ref, b1_ref, w2_ref, b2_ref,
                 pw1_ref, pb1_ref, pw2_ref, pb2_ref, p_ref):
    v = jnp.concatenate([va_ref[...], vb_ref[...]], axis=1)
    h = _ssp(_bdot(v, w1_ref[...]) + b1_ref[...])
    h = _bdot(h, w2_ref[...]) + b2_ref[...]
    x3 = x_ref[...] + h
    p = _ssp(_bdot(x3, pw1_ref[...]) + pb1_ref[...])
    p_ref[...] = _bdot(p, pw2_ref[...]) + pb2_ref[...]


def _head_kernel(p_ref, batch_ref, solv_ref, embs_ref, sw1_ref, sb1_ref,
                 sw2_ref, sb2_ref, qw1a_ref, qw1b_ref, qb1_ref,
                 qw2_ref, qb2_ref, qw3_ref, qb3_ref, out_ref,
                 acc_ref, cnt_ref):
    i = pl.program_id(0)

    @pl.when(i == 0)
    def _init():
        acc_ref[...] = jnp.zeros_like(acc_ref)
        cnt_ref[...] = jnp.zeros_like(cnt_ref)

    onehot = (lax.broadcasted_iota(jnp.int32, (B, 1), 0) == batch_ref[...]
              ).astype(jnp.float32)
    acc_ref[...] += jnp.dot(onehot, p_ref[...], preferred_element_type=jnp.float32,
                            precision=lax.Precision.HIGHEST)
    cnt_ref[...] += jnp.sum(onehot, axis=1, keepdims=True)

    @pl.when(i == NPAD // NB - 1)
    def _final():
        xg = acc_ref[...] / jnp.maximum(cnt_ref[...], 1.0)
        oh = (solv_ref[...] == lax.broadcasted_iota(jnp.int32, (1, 4), 1)
              ).astype(jnp.float32)
        s = jnp.dot(oh, embs_ref[...], preferred_element_type=jnp.float32,
                    precision=lax.Precision.HIGHEST)
        s = _ssp(_bdot(s, sw1_ref[...]) + sb1_ref[...])
        s = _bdot(s, sw2_ref[...]) + sb2_ref[...]
        h = _ssp(_bdot(xg, qw1a_ref[...])
                 + _bdot(s, qw1b_ref[...])
                 + qb1_ref[...])
        h = _ssp(_bdot(h, qw2_ref[...]) + qb2_ref[...])
        out_ref[...] = _bdot(h, qw3_ref[...]) + qb3_ref[...]


def _full(shape):
    return pl.BlockSpec(shape, lambda i: tuple(0 for _ in shape))


_MESH = plsc.VectorSubcoreMesh(core_axis_name="c", subcore_axis_name="s",
                               num_cores=NC, num_subcores=NS)


@functools.partial(
    pl.kernel,
    out_type=jax.ShapeDtypeStruct((EPAD,), jnp.float32),
    mesh=_MESH,
    scratch_types=[
        pltpu.VMEM((2, C), jnp.int32),
        pltpu.VMEM((2, C), jnp.int32),
        pltpu.VMEM((C, 16), jnp.float32),
        pltpu.VMEM((C, 16), jnp.float32),
        pltpu.VMEM((C,), jnp.float32),
        pltpu.SemaphoreType.DMA,
        pltpu.SemaphoreType.DMA,
    ],
    compiler_params=pltpu.CompilerParams(needs_layout_passes=False,
                                         use_tc_tiling_on_sc=False),
)
def _sc_dist(pos_hbm, src_hbm, dst_hbm, d2_hbm,
             sidx, didx, ps, pd, d2v, sem1, sem2):
    c = lax.axis_index("c")
    s = lax.axis_index("s")
    wid = s * NC + c
    per_w = EPAD // (NC * NS)
    nchunks = per_w // C

    def idx_start(j, b):
        base = wid * per_w + j * C
        pltpu.async_copy(src_hbm.at[pl.ds(base, C)], sidx.at[b], sem2)
        pltpu.async_copy(dst_hbm.at[pl.ds(base, C)], didx.at[b], sem2)

    def idx_wait(j, b):
        base = wid * per_w + j * C
        pltpu.make_async_copy(src_hbm.at[pl.ds(base, C)], sidx.at[b], sem2).wait()
        pltpu.make_async_copy(dst_hbm.at[pl.ds(base, C)], didx.at[b], sem2).wait()

    idx_start(0, 0)

    def chunk(j, _):
        b = lax.rem(j, 2)
        base = wid * per_w + j * C
        idx_wait(j, b)
        cp1 = pltpu.async_copy(pos_hbm.at[sidx.at[b]], ps, sem1)
        cp2 = pltpu.async_copy(pos_hbm.at[didx.at[b]], pd, sem1)

        @pl.when(j + 1 < nchunks)
        def _pre():
            idx_start(j + 1, lax.rem(j + 1, 2))

        cp1.wait()
        cp2.wait()
        lanes = lax.iota(jnp.int32, 16)
        for i in range(C // 16):
            rows = lanes + (i * 16)
            c0 = jnp.zeros((16,), jnp.int32)
            c1 = jnp.full((16,), 1, jnp.int32)
            c2 = jnp.full((16,), 2, jnp.int32)
            dx = (plsc.load_gather(ps, [rows, c0])
                  - plsc.load_gather(pd, [rows, c0]))
            dy = (plsc.load_gather(ps, [rows, c1])
                  - plsc.load_gather(pd, [rows, c1]))
            dz = (plsc.load_gather(ps, [rows, c2])
                  - plsc.load_gather(pd, [rows, c2]))
            d2v[pl.ds(i * 16, 16)] = dx * dx + dy * dy + dz * dz
        pltpu.sync_copy(d2v, d2_hbm.at[pl.ds(base, C)])
        return 0

    lax.fori_loop(0, nchunks, chunk, 0)


@functools.partial(
    pl.kernel,
    out_type=[jax.ShapeDtypeStruct((NPAD, 32), jnp.float32),
              jax.ShapeDtypeStruct((NPAD, 32), jnp.float32)],
    mesh=_MESH,
    scratch_types=[
        pltpu.VMEM((2, C), jnp.int32),
        pltpu.VMEM((2, C), jnp.int32),
        pltpu.VMEM((2, C), jnp.int32),
        pltpu.VMEM((2, C), jnp.int32),
        pltpu.VMEM((C, 32), jnp.float32),
        pltpu.VMEM((C, 32), jnp.float32),
        pltpu.VMEM((2 * C, 32), jnp.float32),
        pltpu.VMEM((2 * C, 32), jnp.float32),
        pltpu.VMEM_SHARED((NPAD, 32), jnp.float32),
        pltpu.SemaphoreType.DMA,
        pltpu.SemaphoreType.DMA,
    ],
    compiler_params=pltpu.CompilerParams(use_tc_tiling_on_sc=False),
)
def _sc_msg(ma_hbm, mb_hbm, ga_hbm, gb_hbm, src_hbm, dst_hbm, zeros_hbm,
            va_hbm, vb_hbm,
            sidx0, sidx1, didx0, didx1, rows0, rows1, gv, msg, acc, gsem, isem):
    c = lax.axis_index("c")
    s = lax.axis_index("s")
    stripe = NPAD // NS
    per_s = EPAD // NS
    nchunks = per_s // (2 * C)

    def idx_start(j, b):
        base = s * per_s + j * (2 * C)
        pltpu.async_copy(src_hbm.at[pl.ds(base, C)], sidx0.at[b], isem)
        pltpu.async_copy(src_hbm.at[pl.ds(base + C, C)], sidx1.at[b], isem)
        pltpu.async_copy(dst_hbm.at[pl.ds(base, C)], didx0.at[b], isem)
        pltpu.async_copy(dst_hbm.at[pl.ds(base + C, C)], didx1.at[b], isem)

    def idx_wait(j, b):
        base = s * per_s + j * (2 * C)
        pltpu.make_async_copy(src_hbm.at[pl.ds(base, C)], sidx0.at[b], isem).wait()
        pltpu.make_async_copy(src_hbm.at[pl.ds(base + C, C)], sidx1.at[b], isem).wait()
        pltpu.make_async_copy(dst_hbm.at[pl.ds(base, C)], didx0.at[b], isem).wait()
        pltpu.make_async_copy(dst_hbm.at[pl.ds(base + C, C)], didx1.at[b], isem).wait()

    idx_start(0, 0)
    pltpu.sync_copy(zeros_hbm.at[pl.ds(s * stripe, stripe)],
                    acc.at[pl.ds(s * stripe, stripe)])
    plsc.subcore_barrier()

    def chunk(j, _):
        b = lax.rem(j, 2)
        nb2 = lax.rem(j + 1, 2)
        base = s * per_s + j * (2 * C)
        idx_wait(j, b)

        @pl.when(c == 0)
        def _ca():
            pltpu.async_copy(ma_hbm.at[sidx0.at[b]], rows0, gsem)
            pltpu.async_copy(ma_hbm.at[sidx1.at[b]], rows1, gsem)

        @pl.when(c == 1)
        def _cb():
            pltpu.async_copy(mb_hbm.at[sidx0.at[b]], rows0, gsem)
            pltpu.async_copy(mb_hbm.at[sidx1.at[b]], rows1, gsem)

        @pl.when(j + 1 < nchunks)
        def _pre():
            idx_start(j + 1, nb2)

        @pl.when(c == 0)
        def _ga():
            pltpu.sync_copy(ga_hbm.at[pl.ds(base, 2 * C)], gv)

        @pl.when(c == 1)
        def _gb():
            pltpu.sync_copy(gb_hbm.at[pl.ds(base, 2 * C)], gv)

        pltpu.make_async_copy(ma_hbm.at[sidx0.at[b]], rows0, gsem).wait()
        pltpu.make_async_copy(ma_hbm.at[sidx1.at[b]], rows1, gsem).wait()

        def mrow(i, _):
            msg[i, 0:16] = rows0[i, 0:16] * gv[i, 0:16]
            msg[i, 16:32] = rows0[i, 16:32] * gv[i, 16:32]
            k = i + C
            msg[k, 0:16] = rows1[i, 0:16] * gv[k, 0:16]
            msg[k, 16:32] = rows1[i, 16:32] * gv[k, 16:32]
            return 0

        lax.fori_loop(0, C, mrow, 0, unroll=4)
        pltpu.sync_copy(msg.at[pl.ds(0, C)], acc.at[didx0.at[b]], add=True)
        pltpu.sync_copy(msg.at[pl.ds(C, C)], acc.at[didx1.at[b]], add=True)
        return 0

    lax.fori_loop(0, nchunks, chunk, 0)
    plsc.subcore_barrier()

    @pl.when(c == 0)
    def _oa():
        pltpu.sync_copy(acc.at[pl.ds(s * stripe, stripe)],
                        va_hbm.at[pl.ds(s * stripe, stripe)])

    @pl.when(c == 1)
    def _ob():
        pltpu.sync_copy(acc.at[pl.ds(s * stripe, stripe)],
                        vb_hbm.at[pl.ds(s * stripe, stripe)])


def kernel(position, params, z, edge_index, batch, solvent):
    p = params
    f32 = jnp.float32

    pos_pad = jnp.zeros((NPAD, 16), f32).at[:N, :3].set(position)
    src = edge_index[0].astype(jnp.int32)
    dst = edge_index[1].astype(jnp.int32)
    src_pad = jnp.concatenate([src, jnp.zeros((EPAD - E,), jnp.int32)])
    dst_pad = jnp.concatenate([dst, jnp.full((EPAD - E,), TRASH, jnp.int32)])
    zeros_nodes = jnp.zeros((NPAD, 32), f32)
    z_pad = jnp.concatenate([z.astype(jnp.int32), jnp.zeros((NPAD - N,), jnp.int32)])
    z2d = z_pad.reshape(NPAD, 1)
    batch_row = jnp.concatenate([batch.astype(jnp.int32),
                                 jnp.full((NPAD - N,), B, jnp.int32)]).reshape(1, NPAD)
    solv2d = solvent.astype(jnp.int32).reshape(B, 1)
    u_k = jnp.arange(0.0, 5.0, 0.1, dtype=f32).reshape(1, NG)

    ngrid = NPAD // NB
    egrid = EPAD // EB

    nhalf = pl.BlockSpec((NB, 32), lambda i: (i, 0))
    sds_nh = jax.ShapeDtypeStruct((NPAD, 32), f32)

    x, m1a, m1b = pl.pallas_call(
        _emb_kernel,
        grid=(ngrid,),
        in_specs=[pl.BlockSpec((NB, 1), lambda i: (i, 0)),
                  _full((100, NF)), _full((NF, NF)), _full((1, NF))],
        out_specs=[pl.BlockSpec((NB, NF), lambda i: (i, 0)), nhalf, nhalf],
        out_shape=[jax.ShapeDtypeStruct((NPAD, NF), f32), sds_nh, sds_nh],
    )(z2d, p['emb_z'], p['inter'][0]['lin1_w'],
      p['inter'][0]['lin1_b'].reshape(1, NF))

    d2 = _sc_dist(pos_pad, src_pad, dst_pad)

    wspecs = [_full((1, NG)), _full((NG, NF)), _full((1, NF)), _full((NF, NF)),
              _full((1, NF)), _full((NG, NF)), _full((1, NF)), _full((NF, NF)),
              _full((1, NF))]
    ehalf = pl.BlockSpec((EB, 32), lambda i: (i, 0))
    sds_eh = jax.ShapeDtypeStruct((EPAD, 32), f32)
    g1a, g1b, g2a, g2b = pl.pallas_call(
        _g_kernel,
        grid=(egrid,),
        in_specs=[pl.BlockSpec((EB,), lambda i: (i,))] + wspecs,
        out_specs=[ehalf, ehalf, ehalf, ehalf],
        out_shape=[sds_eh, sds_eh, sds_eh, sds_eh],
    )(d2, u_k,
      p['inter'][0]['g_w1'], p['inter'][0]['g_b1'].reshape(1, NF),
      p['inter'][0]['g_w2'], p['inter'][0]['g_b2'].reshape(1, NF),
      p['inter'][1]['g_w1'], p['inter'][1]['g_b1'].reshape(1, NF),
      p['inter'][1]['g_w2'], p['inter'][1]['g_b2'].reshape(1, NF))

    v1a, v1b = _sc_msg(m1a, m1b, g1a, g1b, src_pad, dst_pad, zeros_nodes)

    lp0, lp1 = p['inter'][0], p['inter'][1]
    x2, m2a, m2b = pl.pallas_call(
        _node_kernel,
        grid=(ngrid,),
        in_specs=[nhalf, nhalf,
                  pl.BlockSpec((NB, NF), lambda i: (i, 0)),
                  _full((NF, NF)), _full((1, NF)), _full((NF, NF)), _full((1, NF)),
                  _full((NF, NF)), _full((1, NF))],
        out_specs=[pl.BlockSpec((NB, NF), lambda i: (i, 0)), nhalf, nhalf],
        out_shape=[jax.ShapeDtypeStruct((NPAD, NF), f32), sds_nh, sds_nh],
    )(v1a, v1b, x, lp0['m_w1'], lp0['m_b1'].reshape(1, NF),
      lp0['m_w2'], lp0['m_b2'].reshape(1, NF),
      lp1['lin1_w'], lp1['lin1_b'].reshape(1, NF))

    v2a, v2b = _sc_msg(m2a, m2b, g2a, g2b, src_pad, dst_pad, zeros_nodes)

    post = pl.pallas_call(
        _post_kernel,
        grid=(ngrid,),
        in_specs=[nhalf, nhalf,
                  pl.BlockSpec((NB, NF), lambda i: (i, 0)),
                  _full((NF, NF)), _full((1, NF)), _full((NF, NF)), _full((1, NF)),
                  _full((NF, NF)), _full((1, NF)), _full((NF, 64)), _full((1, 64))],
        out_specs=pl.BlockSpec((NB, 64), lambda i: (i, 0)),
        out_shape=jax.ShapeDtypeStruct((NPAD, 64), f32),
    )(v2a, v2b, x2, lp1['m_w1'], lp1['m_b1'].reshape(1, NF),
      lp1['m_w2'], lp1['m_b2'].reshape(1, NF),
      p['post_w1'], p['post_b1'].reshape(1, NF),
      p['post_w2'], p['post_b2'].reshape(1, 64))

    out = pl.pallas_call(
        _head_kernel,
        grid=(ngrid,),
        in_specs=[pl.BlockSpec((NB, 64), lambda i: (i, 0)),
                  pl.BlockSpec((1, NB), lambda i: (0, i)),
                  _full((B, 1)), _full((4, 64)), _full((64, 64)), _full((1, 64)),
                  _full((64, 32)), _full((1, 32)),
                  _full((64, 128)), _full((32, 128)), _full((1, 128)),
                  _full((128, 32)), _full((1, 32)), _full((32, 1)), _full((1, 1))],
        out_specs=_full((B, 1)),
        out_shape=jax.ShapeDtypeStruct((B, 1), f32),
        scratch_shapes=[pltpu.VMEM((B, 64), f32), pltpu.VMEM((B, 1), f32)],
    )(post, batch_row, solv2d, p['emb_solv'],
      p['solv_w1'], p['solv_b1'].reshape(1, 64),
      p['solv_w2'], p['solv_b2'].reshape(1, 32),
      p['q_w1'][:64], p['q_w1'][64:], p['q_b1'].reshape(1, 128),
      p['q_w2'], p['q_b2'].reshape(1, 32),
      p['q_w3'], p['q_b3'].reshape(1, 1))

    return out
